# baseline (speedup 1.0000x reference)
import functools

import jax
import jax.numpy as jnp
from jax import lax
from jax.experimental import pallas as pl
from jax.experimental.pallas import tpu as pltpu
from jax.experimental.pallas import tpu_sc as plsc

FEAT_DIM = 128
EXP_DIM = 64
POSE_DIM = 6
N_LMK = 68
POS_FREQ = 4
NEAR = 0.1
FAR = 0.25
DEFORM_SCALE = 0.3
ATTR_SCALE = 0.05

BLK = 2048
NW = 32
CH = 96


def _so3_exp(log_rot, eps=1e-4):
    theta2 = jnp.clip(jnp.sum(log_rot * log_rot, axis=-1), eps)
    theta = jnp.sqrt(theta2)
    fac1 = jnp.sin(theta) / theta
    fac2 = (1.0 - jnp.cos(theta)) / theta2
    x, y, z = log_rot[..., 0], log_rot[..., 1], log_rot[..., 2]
    zz = jnp.zeros_like(x)
    K = jnp.stack([
        jnp.stack([zz, -z, y], axis=-1),
        jnp.stack([z, zz, -x], axis=-1),
        jnp.stack([-y, x, zz], axis=-1),
    ], axis=-2)
    I = jnp.eye(3, dtype=log_rot.dtype)
    return I + fac1[..., None, None] * K + fac2[..., None, None] * (K @ K)


def _pos_embed_rows(xT):
    scaled = jnp.concatenate([xT * (2.0 ** i) for i in range(POS_FREQ)], axis=0)
    return jnp.concatenate([xT, jnp.sin(scaled), jnp.cos(scaled)], axis=0)


def _dist_gate(xyzT, lmk):
    dx = xyzT[0:1, :] - lmk[:, 0:1]
    dy = xyzT[1:2, :] - lmk[:, 1:2]
    dz = xyzT[2:3, :] - lmk[:, 2:2 + 1]
    d2 = dx * dx + dy * dy + dz * dz
    return jnp.min(d2, axis=0, keepdims=True)


def _mask_body(xyzT_ref, lmk_ref, mask_ref):
    dists = _dist_gate(xyzT_ref[...], lmk_ref[...])
    mask_ref[...] = (dists < FAR).astype(jnp.float32)


def _scan_body(mask_ref, ut_ref, sl_ref, pos_ref, nact_ref):
    f32 = jnp.float32
    m = mask_ref[...]
    cum = jnp.dot(m, ut_ref[...], preferred_element_type=f32)
    tot = cum[:, 127:128]
    offs = jnp.dot(sl_ref[...], tot, preferred_element_type=f32)
    cum = cum + offs
    s2 = jnp.sum(jnp.sum(m, axis=0, keepdims=True), axis=1, keepdims=True)
    rows = m.shape[0]
    ii = (lax.broadcasted_iota(jnp.int32, (rows, 128), 0) * 128
          + lax.broadcasted_iota(jnp.int32, (rows, 128), 1)).astype(f32)
    pos = jnp.where(m > 0.5, cum - 1.0, s2 + ii - cum)
    pos_ref[...] = pos.astype(jnp.int32)
    nact_ref[...] = s2.astype(jnp.int32)


def _tmm(w_ref, h, bT_ref):
    out = jax.lax.dot_general(
        w_ref[...], h, (((0,), (1,)), ((), ())),
        preferred_element_type=jnp.float32)
    return out + bT_ref[...]


def _main_body(blk,
               nact_ref,
               featp_ref, smallp_ref,
               lmk_ref, ec_ref, poseT_ref, scale_ref, R9_ref, R6_ref,
               Wf_ref, Wx_ref, Wech_ref, bech_ref, Wpeh_ref, bpeh_ref,
               Wce2_ref, bce2_ref, Wae2_ref, bae2_ref, Wde2_ref, bde2_ref,
               Wce3_ref, bce3T_ref, Wae3_ref, bae3T_ref, Wde3_ref, bde3T_ref,
               Wcp2_ref, bcp2T_ref, Wap2_ref, bap2T_ref, Wdp2_ref, bdp2T_ref,
               out_ref, exp_scr):
    f32 = jnp.float32
    pid = pl.program_id(0)
    active = pid * blk < nact_ref[0]

    smallT = smallp_ref[...][:, 0:16].T
    xyzT = smallT[0:3]
    spT = smallT[3:6]
    rpT = smallT[6:10]
    opT = smallT[10:11]
    feat = jnp.tanh(featp_ref[...])

    dists = _dist_gate(xyzT, lmk_ref[...])
    wT = jnp.clip((FAR - dists) / (FAR - NEAR), 0.0, 1.0)

    embT = _pos_embed_rows(xyzT)
    emb_poseT = _pos_embed_rows(poseT_ref[...])

    relu = lambda v: jnp.maximum(v, 0.0)
    mm = lambda a, w_ref, b_ref: (
        jnp.dot(a, w_ref[...], preferred_element_type=f32) + b_ref[...])

    F1p = jnp.dot(feat, Wf_ref[:, 512:768], preferred_element_type=f32)
    X1p = jax.lax.dot_general(
        embT, Wx_ref[:, 256:384], (((0,), (0,)), ((), ())),
        preferred_element_type=f32)
    ofs_p = jax.lax.dot_general(
        emb_poseT, Wpeh_ref[...], (((0,), (0,)), ((), ())),
        preferred_element_type=f32) + bpeh_ref[...]

    @pl.when(active)
    def _compute_exp():
        F1e = jnp.dot(feat, Wf_ref[:, 0:512], preferred_element_type=f32)
        X1e = jax.lax.dot_general(
            embT, Wx_ref[:, 0:256], (((0,), (0,)), ((), ())),
            preferred_element_type=f32)
        ofs_e = jnp.dot(ec_ref[...], Wech_ref[...],
                        preferred_element_type=f32) + bech_ref[...]
        for b in range(2):
            hc = relu(F1e[:, 0:256] + ofs_e[b:b + 1, 0:256])
            colT_e = _tmm(Wce3_ref, relu(mm(hc, Wce2_ref, bce2_ref)),
                          bce3T_ref)
            ha = relu(F1e[:, 256:512] + ofs_e[b:b + 1, 256:512])
            attrT_e = _tmm(Wae3_ref, relu(mm(ha, Wae2_ref, bae2_ref)),
                           bae3T_ref)
            hd = relu(X1e + ofs_e[b:b + 1, 512:768])
            dxyzT_e = jnp.tanh(_tmm(Wde3_ref,
                                    relu(mm(hd, Wde2_ref, bde2_ref)),
                                    bde3T_ref))
            exp_scr[b] = jnp.concatenate(
                [colT_e, attrT_e, dxyzT_e,
                 jnp.zeros((5, colT_e.shape[1]), f32)], axis=0)

    @pl.when(jnp.logical_not(active))
    def _zero_exp():
        exp_scr[...] = jnp.zeros(exp_scr.shape, f32)

    R9 = R9_ref[...]
    R6 = R6_ref[...]
    scv = scale_ref[...]
    poseT = poseT_ref[...]
    pw = 1.0 - wT

    for b in range(2):
        es = exp_scr[b]
        hpc = relu(F1p[:, 0:128] + ofs_p[b:b + 1, 0:128])
        colT_p = _tmm(Wcp2_ref, hpc, bcp2T_ref)
        hpa = relu(F1p[:, 128:256] + ofs_p[b:b + 1, 128:256])
        attrT_p = _tmm(Wap2_ref, hpa, bap2T_ref)
        hpd = relu(X1p + ofs_p[b:b + 1, 256:384])
        dxyzT_p = jnp.tanh(_tmm(Wdp2_ref, hpd, bdp2T_ref))

        color = es[0:32] * wT + colT_p * pw
        daT = es[32:40] * wT + attrT_p * pw
        dxyzT = es[40:43] * wT + dxyzT_p * pw

        S = scv[b, 0]
        sclT = jnp.exp(spT + daT[0:3] * ATTR_SCALE) * S
        opaT = jax.nn.sigmoid(opT + daT[7:8] * ATTR_SCALE)

        xsT = (xyzT + dxyzT * DEFORM_SCALE) * S
        Rb = R6[3 * b:3 * b + 3, :]
        xyzoT = (jnp.dot(Rb, xsT, preferred_element_type=f32)
                 + poseT[3:6, b:b + 1])

        q = rpT + daT[3:7] * ATTR_SCALE
        qn = q / jnp.maximum(
            jnp.sqrt(jnp.sum(q * q, axis=0, keepdims=True)), 1e-12)
        r = qn[0:1]; i_ = qn[1:2]; j_ = qn[2:3]; k_ = qn[3:4]
        two_s = 2.0 / jnp.sum(qn * qn, axis=0, keepdims=True)
        M = [[1 - two_s * (j_ * j_ + k_ * k_), two_s * (i_ * j_ - k_ * r),
              two_s * (i_ * k_ + j_ * r)],
             [two_s * (i_ * j_ + k_ * r), 1 - two_s * (i_ * i_ + k_ * k_),
              two_s * (j_ * k_ - i_ * r)],
             [two_s * (i_ * k_ - j_ * r), two_s * (j_ * k_ + i_ * r),
              1 - two_s * (i_ * i_ + j_ * j_)]]
        rm = [[R9[b, 3 * a_ + 0] * M[0][c_] + R9[b, 3 * a_ + 1] * M[1][c_]
               + R9[b, 3 * a_ + 2] * M[2][c_]
               for c_ in range(3)] for a_ in range(3)]
        m00, m01, m02 = rm[0]
        m10, m11, m12 = rm[1]
        m20, m21, m22 = rm[2]
        s0 = 1.0 + m00 + m11 + m22
        s1 = 1.0 + m00 - m11 - m22
        s2 = 1.0 - m00 + m11 - m22
        s3 = 1.0 - m00 - m11 + m22
        qa = [jnp.sqrt(jnp.maximum(s_, 1e-8)) for s_ in (s0, s1, s2, s3)]
        cands = [
            [qa[0] * qa[0], m21 - m12, m02 - m20, m10 - m01],
            [m21 - m12, qa[1] * qa[1], m10 + m01, m02 + m20],
            [m02 - m20, m10 + m01, qa[2] * qa[2], m12 + m21],
            [m10 - m01, m20 + m02, m21 + m12, qa[3] * qa[3]],
        ]
        mx = jnp.maximum(jnp.maximum(qa[0], qa[1]), jnp.maximum(qa[2], qa[3]))
        isel = [(qa_k >= mx).astype(f32) for qa_k in qa]
        f_sel = [isel[0],
                 isel[1] * (1.0 - isel[0]),
                 isel[2] * (1.0 - isel[0]) * (1.0 - isel[1]),
                 isel[3] * (1.0 - isel[0]) * (1.0 - isel[1]) * (1.0 - isel[2])]
        rows = []
        for c_ in range(4):
            acc = f_sel[0] * cands[0][c_]
            for k2 in range(1, 4):
                acc = acc + f_sel[k2] * cands[k2][c_]
            den = (f_sel[0] * (2.0 * jnp.maximum(qa[0], 0.1))
                   + f_sel[1] * (2.0 * jnp.maximum(qa[1], 0.1))
                   + f_sel[2] * (2.0 * jnp.maximum(qa[2], 0.1))
                   + f_sel[3] * (2.0 * jnp.maximum(qa[3], 0.1)))
            rows.append(acc / den)
        rotT = jnp.concatenate(rows, axis=0)

        outT = jnp.concatenate(
            [xyzoT, color, sclT, rotT, opaT,
             jnp.zeros((5, blk), f32)], axis=0)
        out_ref[:, 48 * b:48 * b + 48] = outT.T
    out_ref[:, 96:128] = jnp.zeros((blk, 32), f32)


def _sc_scatter_inputs(feature, small, pos3, Np):
    f32 = jnp.float32
    iters = Np // (NW * CH)

    @functools.partial(
        pl.kernel,
        mesh=plsc.VectorSubcoreMesh(core_axis_name="c", subcore_axis_name="s"),
        out_type=[jax.ShapeDtypeStruct((Np, 128), f32),
                  jax.ShapeDtypeStruct((Np, 128), f32)],
        scratch_types=[pltpu.VMEM((iters, CH), jnp.int32)]
        + [pltpu.VMEM((CH, 128), f32) for _ in range(6)]
        + [pltpu.SemaphoreType.DMA for _ in range(12)],
    )
    def k(feat_hbm, small_hbm, pos_hbm, fout_hbm, sout_hbm,
          idx_v, fb0, fb1, fb2, sb0, sb1, sb2, *sems):
        rsem = sems[0:3]
        wfsem = sems[3:6]
        wssem = sems[6:9]
        rssem = sems[9:12]
        fbufs = [fb0, fb1, fb2]
        sbufs = [sb0, sb1, sb2]
        wid = lax.axis_index("s") * 2 + lax.axis_index("c")
        base = wid * (iters * CH)
        pltpu.sync_copy(pos_hbm.at[wid], idx_v)
        rd = {}

        def issue_read(j):
            b = j % 3
            rd[j] = (
                pltpu.async_copy(feat_hbm.at[pl.ds(base + j * CH, CH)],
                                 fbufs[b], rsem[b]),
                pltpu.async_copy(small_hbm.at[pl.ds(base + j * CH, CH)],
                                 sbufs[b], rssem[b]))

        for j in range(min(2, iters)):
            issue_read(j)
        wd = {}
        for j in range(iters):
            b = j % 3
            rd[j][0].wait()
            rd[j][1].wait()
            wd[j] = (
                pltpu.async_copy(fbufs[b], fout_hbm.at[idx_v.at[j]],
                                 wfsem[b]),
                pltpu.async_copy(sbufs[b], sout_hbm.at[idx_v.at[j]],
                                 wssem[b]))
            if j - 1 >= 0:
                wd[j - 1][0].wait()
                wd[j - 1][1].wait()
            if j + 2 < iters:
                issue_read(j + 2)
        wd[iters - 1][0].wait()
        wd[iters - 1][1].wait()

    return k(feature, small, pos3)


def _sc_gather_rows(rows, pos3, Np):
    f32 = jnp.float32
    iters = Np // (NW * CH)

    @functools.partial(
        pl.kernel,
        mesh=plsc.VectorSubcoreMesh(core_axis_name="c", subcore_axis_name="s"),
        out_type=jax.ShapeDtypeStruct((Np, 128), f32),
        scratch_types=[pltpu.VMEM((iters, CH), jnp.int32)]
        + [pltpu.VMEM((CH, 128), f32) for _ in range(3)]
        + [pltpu.SemaphoreType.DMA for _ in range(3)],
    )
    def k(rows_hbm, pos_hbm, out_hbm, idx_v, b0, b1, b2, g0, g1, g2):
        bufs = [b0, b1, b2]
        gsem = [g0, g1, g2]
        wid = lax.axis_index("s") * 2 + lax.axis_index("c")
        base = wid * (iters * CH)
        pltpu.sync_copy(pos_hbm.at[wid], idx_v)
        gd = {}
        for j in range(min(3, iters)):
            gd[j] = pltpu.async_copy(rows_hbm.at[idx_v.at[j]], bufs[j % 3],
                                     gsem[j % 3])
        for j in range(iters):
            b = j % 3
            gd[j].wait()
            pltpu.sync_copy(bufs[b], out_hbm.at[pl.ds(base + j * CH, CH)])
            if j + 3 < iters:
                gd[j + 3] = pltpu.async_copy(rows_hbm.at[idx_v.at[j + 3]],
                                             bufs[b], gsem[b])

    return k(rows, pos3)


def kernel(exp_coeff, pose, scale, params, xyz, feature, scales_param,
           rotation_param, opacity_param, landmarks):
    f32 = jnp.float32
    i32 = jnp.int32
    N = xyz.shape[0]
    blk = BLK
    align = NW * CH
    Np = ((N + align - 1) // align) * align
    nblk = Np // blk

    def padT(a):
        aT = a.T
        if Np != N:
            aT = jnp.concatenate(
                [aT, jnp.zeros((aT.shape[0], Np - N), f32)], axis=1)
        return aT

    def padR(a):
        if Np != N:
            a = jnp.concatenate([a, jnp.zeros((Np - N, a.shape[1]), f32)],
                                axis=0)
        return a

    xyzT = padT(xyz)
    featp = padR(feature)
    small = padR(jnp.concatenate(
        [xyz, scales_param, rotation_param, opacity_param,
         jnp.zeros((N, 117), f32)], axis=1))

    pc, pa, pd = params["exp_color"], params["exp_attributes"], params["exp_deform"]
    qc, qa_, qd = params["pose_color"], params["pose_attributes"], params["pose_deform"]
    W_feat = jnp.concatenate([pc["w"][0][:FEAT_DIM], pa["w"][0][:FEAT_DIM],
                              qc["w"][0][:FEAT_DIM], qa_["w"][0][:FEAT_DIM]],
                             axis=1)
    perm = ([0, 1, 2] + [3 + 6 * i + j for i in range(POS_FREQ) for j in range(3)]
            + [6 + 6 * i + j for i in range(POS_FREQ) for j in range(3)])
    W_xyz = jnp.concatenate([pd["w"][0][:27][jnp.array(perm)],
                             qd["w"][0][:27][jnp.array(perm)]], axis=1)
    Wec_hi = jnp.concatenate([pc["w"][0][FEAT_DIM:], pa["w"][0][FEAT_DIM:],
                              pd["w"][0][27:]], axis=1)
    bec = jnp.concatenate([pc["b"][0], pa["b"][0], pd["b"][0]])[None]
    perm54 = ([0, 1, 2, 3, 4, 5]
              + [6 + 12 * i + j for i in range(POS_FREQ) for j in range(6)]
              + [12 + 12 * i + j for i in range(POS_FREQ) for j in range(6)])
    Wpe_hi = jnp.concatenate([qc["w"][0][FEAT_DIM:], qa_["w"][0][FEAT_DIM:],
                              qd["w"][0][27:]], axis=1)[jnp.array(perm54)]
    bpe = jnp.concatenate([qc["b"][0], qa_["b"][0], qd["b"][0]])[None]

    Rm = _so3_exp(pose[:, :3])
    R9 = Rm.reshape(2, 9)
    R6 = jnp.concatenate([Rm[0], Rm[1]], axis=0)

    rep = lambda s: pl.BlockSpec(s, lambda *a: (0,) * len(s))

    mask = pl.pallas_call(
        _mask_body,
        grid=(nblk,),
        in_specs=[pl.BlockSpec((3, blk), lambda i, *_: (0, i)),
                  rep((N_LMK, 3))],
        out_specs=pl.BlockSpec((1, blk), lambda i, *_: (0, i)),
        out_shape=jax.ShapeDtypeStruct((1, Np), f32),
    )(xyzT, landmarks)

    rows = Np // 128
    ut = jnp.triu(jnp.ones((128, 128), f32))
    sl = jnp.tril(jnp.ones((rows, rows), f32), k=-1)
    pos, nact = pl.pallas_call(
        _scan_body,
        in_specs=[rep((rows, 128)), rep((128, 128)), rep((rows, rows))],
        out_specs=[rep((rows, 128)), rep((1, 1))],
        out_shape=[jax.ShapeDtypeStruct((rows, 128), i32),
                   jax.ShapeDtypeStruct((1, 1), i32)],
    )(mask.reshape(rows, 128), ut, sl)
    pos3 = pos.reshape(NW, Np // (NW * CH), CH)
    nact1 = nact.reshape((1,))

    featp_perm, small_perm = _sc_scatter_inputs(featp, small, pos3, Np)

    in_specs = [
        pl.BlockSpec((blk, FEAT_DIM), lambda i, *_: (i, 0)),
        pl.BlockSpec((blk, 128), lambda i, *_: (i, 0)),
        rep((N_LMK, 3)), rep((2, EXP_DIM)), rep((POSE_DIM, 2)),
        rep((2, 1)), rep((2, 9)), rep((6, 3)),
        rep((FEAT_DIM, 768)), rep((27, 384)), rep((EXP_DIM, 768)),
        rep((1, 768)), rep((54, 384)), rep((1, 384)),
        rep((256, 256)), rep((1, 256)), rep((256, 256)), rep((1, 256)),
        rep((256, 256)), rep((1, 256)),
        rep((256, 32)), rep((32, 1)), rep((256, 8)), rep((8, 1)),
        rep((256, 3)), rep((3, 1)),
        rep((128, 32)), rep((32, 1)), rep((128, 8)), rep((8, 1)),
        rep((128, 3)), rep((3, 1)),
    ]
    bT = lambda v: v[:, None]
    operands = [
        featp_perm, small_perm,
        landmarks, exp_coeff, pose.T, scale, R9, R6,
        W_feat, W_xyz, Wec_hi, bec, Wpe_hi, bpe,
        pc["w"][1], pc["b"][1][None], pa["w"][1], pa["b"][1][None],
        pd["w"][1], pd["b"][1][None],
        pc["w"][2], bT(pc["b"][2]), pa["w"][2], bT(pa["b"][2]),
        pd["w"][2], bT(pd["b"][2]),
        qc["w"][1], bT(qc["b"][1]), qa_["w"][1], bT(qa_["b"][1]),
        qd["w"][1], bT(qd["b"][1]),
    ]
    outp = pl.pallas_call(
        functools.partial(_main_body, blk),
        grid_spec=pltpu.PrefetchScalarGridSpec(
            num_scalar_prefetch=1,
            grid=(nblk,),
            in_specs=in_specs,
            out_specs=pl.BlockSpec((blk, 128), lambda i, *_: (i, 0)),
            scratch_shapes=[pltpu.VMEM((2, 48, blk), f32)],
        ),
        out_shape=jax.ShapeDtypeStruct((Np, 128), f32),
    )(nact1, *operands)

    final = _sc_gather_rows(outp, pos3, Np)

    def leaf(off, c):
        return jnp.stack([final[:N, off:off + c],
                          final[:N, 48 + off:48 + off + c]], axis=0)

    xyz_out = leaf(0, 3)
    color = leaf(3, 32)
    scales = leaf(35, 3)
    rotation = leaf(38, 4)
    opacity = leaf(42, 1)
    return xyz_out, color, scales, rotation, opacity


# speedup vs baseline: 2.6550x; 2.6550x over previous
import functools

import jax
import jax.numpy as jnp
from jax.experimental import pallas as pl

FEAT_DIM = 128
EXP_DIM = 64
POSE_DIM = 6
N_LMK = 68
POS_FREQ = 4
NEAR = 0.1
FAR = 0.25
DEFORM_SCALE = 0.3
ATTR_SCALE = 0.05

BLK = 6144


def _so3_exp(log_rot, eps=1e-4):
    theta2 = jnp.clip(jnp.sum(log_rot * log_rot, axis=-1), eps)
    theta = jnp.sqrt(theta2)
    fac1 = jnp.sin(theta) / theta
    fac2 = (1.0 - jnp.cos(theta)) / theta2
    x, y, z = log_rot[..., 0], log_rot[..., 1], log_rot[..., 2]
    zz = jnp.zeros_like(x)
    K = jnp.stack([
        jnp.stack([zz, -z, y], axis=-1),
        jnp.stack([z, zz, -x], axis=-1),
        jnp.stack([-y, x, zz], axis=-1),
    ], axis=-2)
    I = jnp.eye(3, dtype=log_rot.dtype)
    return I + fac1[..., None, None] * K + fac2[..., None, None] * (K @ K)


def _pos_embed_rows(xT):
    scaled = jnp.concatenate([xT * (2.0 ** i) for i in range(POS_FREQ)], axis=0)
    return jnp.concatenate([xT, jnp.sin(scaled), jnp.cos(scaled)], axis=0)


def _tmm(w_ref, h, bT_ref):
    out = jax.lax.dot_general(
        w_ref[...], h, (((0,), (1,)), ((), ())),
        preferred_element_type=jnp.float32)
    return out + bT_ref[...]


def _body(blk,
          xyzT_ref, feat_ref, spT_ref, rpT_ref, opT_ref,
          lmk_ref, ec_ref, pose_ref, poseT_ref, scale_ref, R9_ref, R6_ref,
          Wf_ref, Wx_ref, Wech_ref, bech_ref, Wpeh_ref, bpeh_ref,
          Wce2_ref, bce2_ref, Wae2_ref, bae2_ref, Wde2_ref, bde2_ref,
          Wce3_ref, bce3T_ref, Wae3_ref, bae3T_ref, Wde3_ref, bde3T_ref,
          Wcp2_ref, bcp2T_ref, Wap2_ref, bap2T_ref, Wdp2_ref, bdp2T_ref,
          oxyz_ref, ocol_ref, oscl_ref, orot_ref, oopa_ref):
    f32 = jnp.float32
    xyzT = xyzT_ref[...]
    feat = jnp.tanh(feat_ref[...])

    lmk = lmk_ref[...]
    dx = xyzT[0:1, :] - lmk[:, 0:1]
    dy = xyzT[1:2, :] - lmk[:, 1:2]
    dz = xyzT[2:3, :] - lmk[:, 2:3]
    d2 = dx * dx + dy * dy + dz * dz
    dists = jnp.min(d2, axis=0, keepdims=True)
    wT = jnp.clip((FAR - dists) / (FAR - NEAR), 0.0, 1.0)

    embT = _pos_embed_rows(xyzT)
    emb_poseT = _pos_embed_rows(poseT_ref[...])

    F1 = jnp.dot(feat, Wf_ref[...], preferred_element_type=f32)
    X1 = jax.lax.dot_general(
        embT, Wx_ref[...], (((0,), (0,)), ((), ())),
        preferred_element_type=f32)

    ofs_e = jnp.dot(ec_ref[...], Wech_ref[...],
                    preferred_element_type=f32) + bech_ref[...]
    ofs_p = jax.lax.dot_general(
        emb_poseT, Wpeh_ref[...], (((0,), (0,)), ((), ())),
        preferred_element_type=f32) + bpeh_ref[...]

    relu = lambda v: jnp.maximum(v, 0.0)
    mm = lambda a, w_ref, b_ref: (
        jnp.dot(a, w_ref[...], preferred_element_type=f32) + b_ref[...])

    R9 = R9_ref[...]
    R6 = R6_ref[...]
    scv = scale_ref[...]
    poseT = poseT_ref[...]
    spT = spT_ref[...]
    rpT = rpT_ref[...]
    opT = opT_ref[...]

    for b in range(2):
        hc = relu(F1[:, 0:256] + ofs_e[b:b + 1, 0:256])
        colT_e = _tmm(Wce3_ref, relu(mm(hc, Wce2_ref, bce2_ref)),
                      bce3T_ref)
        ha = relu(F1[:, 256:512] + ofs_e[b:b + 1, 256:512])
        attrT_e = _tmm(Wae3_ref, relu(mm(ha, Wae2_ref, bae2_ref)),
                       bae3T_ref)
        hd = relu(X1[:, 0:256] + ofs_e[b:b + 1, 512:768])
        dxyzT_e = jnp.tanh(_tmm(Wde3_ref, relu(mm(hd, Wde2_ref, bde2_ref)),
                                bde3T_ref))
        hpc = relu(F1[:, 512:640] + ofs_p[b:b + 1, 0:128])
        colT_p = _tmm(Wcp2_ref, hpc, bcp2T_ref)
        hpa = relu(F1[:, 640:768] + ofs_p[b:b + 1, 128:256])
        attrT_p = _tmm(Wap2_ref, hpa, bap2T_ref)
        hpd = relu(X1[:, 256:384] + ofs_p[b:b + 1, 256:384])
        dxyzT_p = jnp.tanh(_tmm(Wdp2_ref, hpd, bdp2T_ref))

        pw = 1.0 - wT
        ocol_ref[b] = colT_e * wT + colT_p * pw
        daT = attrT_e * wT + attrT_p * pw
        dxyzT = dxyzT_e * wT + dxyzT_p * pw

        S = scv[b, 0]
        oscl_ref[b] = jnp.exp(spT + daT[0:3] * ATTR_SCALE) * S
        oopa_ref[b] = jax.nn.sigmoid(opT + daT[7:8] * ATTR_SCALE)

        xsT = (xyzT + dxyzT * DEFORM_SCALE) * S
        Rb = R6[3 * b:3 * b + 3, :]
        oxyz_ref[b] = (jnp.dot(Rb, xsT, preferred_element_type=f32)
                       + poseT[3:6, b:b + 1])

        q = rpT + daT[3:7] * ATTR_SCALE
        qn = q / jnp.maximum(
            jnp.sqrt(jnp.sum(q * q, axis=0, keepdims=True)), 1e-12)
        r = qn[0:1]; i_ = qn[1:2]; j_ = qn[2:3]; k_ = qn[3:4]
        two_s = 2.0 / jnp.sum(qn * qn, axis=0, keepdims=True)
        M = [[1 - two_s * (j_ * j_ + k_ * k_), two_s * (i_ * j_ - k_ * r),
              two_s * (i_ * k_ + j_ * r)],
             [two_s * (i_ * j_ + k_ * r), 1 - two_s * (i_ * i_ + k_ * k_),
              two_s * (j_ * k_ - i_ * r)],
             [two_s * (i_ * k_ - j_ * r), two_s * (j_ * k_ + i_ * r),
              1 - two_s * (i_ * i_ + j_ * j_)]]
        rm = [[R9[b, 3 * a_ + 0] * M[0][c_] + R9[b, 3 * a_ + 1] * M[1][c_]
               + R9[b, 3 * a_ + 2] * M[2][c_]
               for c_ in range(3)] for a_ in range(3)]
        m00, m01, m02 = rm[0]
        m10, m11, m12 = rm[1]
        m20, m21, m22 = rm[2]
        s0 = 1.0 + m00 + m11 + m22
        s1 = 1.0 + m00 - m11 - m22
        s2 = 1.0 - m00 + m11 - m22
        s3 = 1.0 - m00 - m11 + m22
        qa = [jnp.sqrt(jnp.maximum(s_, 1e-8)) for s_ in (s0, s1, s2, s3)]
        cands = [
            [qa[0] * qa[0], m21 - m12, m02 - m20, m10 - m01],
            [m21 - m12, qa[1] * qa[1], m10 + m01, m02 + m20],
            [m02 - m20, m10 + m01, qa[2] * qa[2], m12 + m21],
            [m10 - m01, m20 + m02, m21 + m12, qa[3] * qa[3]],
        ]
        mx = jnp.maximum(jnp.maximum(qa[0], qa[1]), jnp.maximum(qa[2], qa[3]))
        isel = [(qa_k >= mx).astype(f32) for qa_k in qa]
        f_sel = [isel[0],
                 isel[1] * (1.0 - isel[0]),
                 isel[2] * (1.0 - isel[0]) * (1.0 - isel[1]),
                 isel[3] * (1.0 - isel[0]) * (1.0 - isel[1]) * (1.0 - isel[2])]
        rows = []
        for c_ in range(4):
            acc = f_sel[0] * cands[0][c_]
            for k2 in range(1, 4):
                acc = acc + f_sel[k2] * cands[k2][c_]
            den = (f_sel[0] * (2.0 * jnp.maximum(qa[0], 0.1))
                   + f_sel[1] * (2.0 * jnp.maximum(qa[1], 0.1))
                   + f_sel[2] * (2.0 * jnp.maximum(qa[2], 0.1))
                   + f_sel[3] * (2.0 * jnp.maximum(qa[3], 0.1)))
            rows.append(acc / den)
        orot_ref[b] = jnp.concatenate(rows, axis=0)


def kernel(exp_coeff, pose, scale, params, xyz, feature, scales_param,
           rotation_param, opacity_param, landmarks):
    f32 = jnp.float32
    N = xyz.shape[0]
    blk = min(BLK, N)
    Np = ((N + blk - 1) // blk) * blk
    nblk = Np // blk

    def padT(a):
        aT = a.T
        if Np != N:
            aT = jnp.concatenate(
                [aT, jnp.zeros((aT.shape[0], Np - N), f32)], axis=1)
        return aT

    def padR(a):
        if Np != N:
            a = jnp.concatenate([a, jnp.zeros((Np - N, a.shape[1]), f32)],
                                axis=0)
        return a

    pc, pa, pd = params["exp_color"], params["exp_attributes"], params["exp_deform"]
    qc, qa_, qd = params["pose_color"], params["pose_attributes"], params["pose_deform"]
    W_feat = jnp.concatenate([pc["w"][0][:FEAT_DIM], pa["w"][0][:FEAT_DIM],
                              qc["w"][0][:FEAT_DIM], qa_["w"][0][:FEAT_DIM]],
                             axis=1)
    perm = ([0, 1, 2] + [3 + 6 * i + j for i in range(POS_FREQ) for j in range(3)]
            + [6 + 6 * i + j for i in range(POS_FREQ) for j in range(3)])
    W_xyz = jnp.concatenate([pd["w"][0][:27][jnp.array(perm)],
                             qd["w"][0][:27][jnp.array(perm)]], axis=1)
    Wec_hi = jnp.concatenate([pc["w"][0][FEAT_DIM:], pa["w"][0][FEAT_DIM:],
                              pd["w"][0][27:]], axis=1)
    bec = jnp.concatenate([pc["b"][0], pa["b"][0], pd["b"][0]])[None]
    perm54 = ([0, 1, 2, 3, 4, 5]
              + [6 + 12 * i + j for i in range(POS_FREQ) for j in range(6)]
              + [12 + 12 * i + j for i in range(POS_FREQ) for j in range(6)])
    Wpe_hi = jnp.concatenate([qc["w"][0][FEAT_DIM:], qa_["w"][0][FEAT_DIM:],
                              qd["w"][0][27:]], axis=1)[jnp.array(perm54)]
    bpe = jnp.concatenate([qc["b"][0], qa_["b"][0], qd["b"][0]])[None]

    Rm = _so3_exp(pose[:, :3])
    R9 = Rm.reshape(2, 9)
    R6 = jnp.concatenate([Rm[0], Rm[1]], axis=0)

    rep = lambda s: pl.BlockSpec(s, lambda i: (0,) * len(s))
    colr = lambda c: pl.BlockSpec((c, blk), lambda i: (0, i))
    outr = lambda c: pl.BlockSpec((2, c, blk), lambda i: (0, 0, i))

    bT = lambda v: v[:, None]

    operands = [
        padT(xyz), padR(feature), padT(scales_param), padT(rotation_param),
        padT(opacity_param),
        landmarks, exp_coeff, pose, pose.T, scale, R9, R6,
        W_feat, W_xyz, Wec_hi, bec, Wpe_hi, bpe,
        pc["w"][1], pc["b"][1][None], pa["w"][1], pa["b"][1][None],
        pd["w"][1], pd["b"][1][None],
        pc["w"][2], bT(pc["b"][2]), pa["w"][2], bT(pa["b"][2]),
        pd["w"][2], bT(pd["b"][2]),
        qc["w"][1], bT(qc["b"][1]), qa_["w"][1], bT(qa_["b"][1]),
        qd["w"][1], bT(qd["b"][1]),
    ]
    in_specs = [
        colr(3), pl.BlockSpec((blk, FEAT_DIM), lambda i: (i, 0)),
        colr(3), colr(4), colr(1),
        rep((N_LMK, 3)), rep((2, EXP_DIM)), rep((2, POSE_DIM)),
        rep((POSE_DIM, 2)), rep((2, 1)), rep((2, 9)), rep((6, 3)),
        rep((FEAT_DIM, 768)), rep((27, 384)), rep((EXP_DIM, 768)),
        rep((1, 768)), rep((54, 384)), rep((1, 384)),
        rep((256, 256)), rep((1, 256)), rep((256, 256)), rep((1, 256)),
        rep((256, 256)), rep((1, 256)),
        rep((256, 32)), rep((32, 1)), rep((256, 8)), rep((8, 1)),
        rep((256, 3)), rep((3, 1)),
        rep((128, 32)), rep((32, 1)), rep((128, 8)), rep((8, 1)),
        rep((128, 3)), rep((3, 1)),
    ]
    out_shape = [
        jax.ShapeDtypeStruct((2, 3, Np), f32),
        jax.ShapeDtypeStruct((2, 32, Np), f32),
        jax.ShapeDtypeStruct((2, 3, Np), f32),
        jax.ShapeDtypeStruct((2, 4, Np), f32),
        jax.ShapeDtypeStruct((2, 1, Np), f32),
    ]
    out_specs = [outr(3), outr(32), outr(3), outr(4), outr(1)]

    outs = pl.pallas_call(
        functools.partial(_body, blk),
        grid=(nblk,),
        in_specs=in_specs,
        out_specs=out_specs,
        out_shape=out_shape,
    )(*operands)
    xyz_out, color, scales, rotation, opacity = (
        jnp.swapaxes(o, 1, 2)[:, :N] for o in outs)
    return xyz_out, color, scales, rotation, opacity
